# baseline (speedup 1.0000x reference)
import functools

import jax
import jax.numpy as jnp
from jax import lax
from jax.experimental import pallas as pl
from jax.experimental.pallas import tpu as pltpu, tpu_sc as plsc

N = 10000
E = 160000
D = 256
DH = 128
NC, NS = 2, 16
N_PAD = 10240
ROWS = N_PAD // NS
EPT = E // NS
CW = 80
NCH = EPT // CW
GC = 25
NG = NCH // GC
EPW = E // (NC * NS)
CWD = 40
NCHD = EPW // CWD
K = (N + 1) // 2


def _sc_mesh():
    return plsc.VectorSubcoreMesh(core_axis_name="c", subcore_axis_name="s",
                                  num_cores=NC, num_subcores=NS)


def _deg_body(dst_hbm, ones_hbm, zeros_hbm, out0_hbm, out1_hbm,
              dst_ref, ones_ref, sem, acc):
    c = lax.axis_index("c")
    s = lax.axis_index("s")
    w = c * NS + s
    pltpu.sync_copy(dst_hbm.at[w], dst_ref)
    pltpu.sync_copy(ones_hbm, ones_ref)
    pltpu.sync_copy(zeros_hbm, acc.at[pl.ds(s * ROWS, ROWS)])
    plsc.subcore_barrier()

    WIN = 8

    def step(j, carry):
        pltpu.async_copy(ones_ref, acc.at[dst_ref.at[j]], sem, add=True)

        @pl.when(j >= WIN)
        def _():
            pltpu.make_async_copy(ones_ref, acc.at[dst_ref.at[j - WIN]],
                                  sem).wait()

        return carry

    lax.fori_loop(0, NCHD, step, 0)

    def drain(j, carry):
        pltpu.make_async_copy(ones_ref, acc.at[dst_ref.at[j]], sem).wait()
        return carry

    lax.fori_loop(0, WIN, drain, 0)
    plsc.subcore_barrier()

    @pl.when(c == 0)
    def _():
        pltpu.sync_copy(acc.at[pl.ds(s * ROWS, ROWS)],
                        out0_hbm.at[pl.ds(s * ROWS, ROWS)])

    @pl.when(c == 1)
    def _():
        pltpu.sync_copy(acc.at[pl.ds(s * ROWS, ROWS)],
                        out1_hbm.at[pl.ds(s * ROWS, ROWS)])


def _deg_kernel(dst_w, ones_d, zeros_d):
    return pl.kernel(
        _deg_body,
        out_type=(jax.ShapeDtypeStruct((N_PAD, DH), jnp.float32),
                  jax.ShapeDtypeStruct((N_PAD, DH), jnp.float32)),
        mesh=_sc_mesh(),
        scratch_types=[
            pltpu.VMEM((NCHD, CWD), jnp.int32),
            pltpu.VMEM((CWD, DH), jnp.float32),
            pltpu.SemaphoreType.DMA,
            pltpu.VMEM_SHARED((N_PAD, DH), jnp.float32),
        ],
    )(dst_w, ones_d, zeros_d)


def _agg_body(plo_hbm, phi_hbm, src_hbm, dst_hbm, out_lo_hbm, out_hi_hbm,
              src_ref, dst_ref, buf0, buf1, buf2,
              gsem0, gsem1, gsem2, ssem0, ssem1, ssem2, acc):
    c = lax.axis_index("c")
    s = lax.axis_index("s")
    bufs = (buf0, buf1, buf2)
    gsems = (gsem0, gsem1, gsem2)
    ssems = (ssem0, ssem1, ssem2)

    def run(p_hbm, out_hbm):
        pltpu.sync_copy(p_hbm.at[pl.ds(s * ROWS, ROWS)],
                        acc.at[pl.ds(s * ROWS, ROWS)])
        plsc.subcore_barrier()

        def fire_g(l, b):
            pltpu.async_copy(p_hbm.at[src_ref.at[l]], bufs[b], gsems[b])

        def wait_g(b):
            pltpu.make_async_copy(p_hbm.at[pl.ds(0, CW)], bufs[b],
                                  gsems[b]).wait()

        def fire_s(l, b):
            pltpu.async_copy(bufs[b], acc.at[dst_ref.at[l]], ssems[b],
                             add=True)

        def wait_s(l, b):
            pltpu.make_async_copy(bufs[b], acc.at[dst_ref.at[l]],
                                  ssems[b]).wait()

        def group(g, carry):
            pltpu.sync_copy(src_hbm.at[s * NG + g], src_ref)
            pltpu.sync_copy(dst_hbm.at[s * NG + g], dst_ref)
            for b in range(3):
                fire_g(b, b)

            def triple(j, c2):
                a = 3 * j
                for b in range(3):
                    wait_g(b)
                    fire_s(a + b, b)
                for b in range(3):
                    wait_s(a + b, b)
                    fire_g(a + 3 + b, b)
                return c2

            lax.fori_loop(0, 7, triple, 0)
            for b in range(3):
                wait_g(b)
                fire_s(21 + b, b)
            wait_s(21, 0)
            fire_g(24, 0)
            wait_g(0)
            fire_s(24, 0)
            wait_s(22, 1)
            wait_s(23, 2)
            wait_s(24, 0)
            return carry

        lax.fori_loop(0, NG, group, 0)
        plsc.subcore_barrier()
        pltpu.sync_copy(acc.at[pl.ds(s * ROWS, ROWS)],
                        out_hbm.at[pl.ds(s * ROWS, ROWS)])

    @pl.when(c == 0)
    def _():
        run(plo_hbm, out_lo_hbm)

    @pl.when(c == 1)
    def _():
        run(phi_hbm, out_hi_hbm)


def _agg_kernel(p_lo, p_hi, src_r, dst_r):
    return pl.kernel(
        _agg_body,
        out_type=(jax.ShapeDtypeStruct((N_PAD, DH), jnp.float32),
                  jax.ShapeDtypeStruct((N_PAD, DH), jnp.float32)),
        mesh=_sc_mesh(),
        scratch_types=[
            pltpu.VMEM((GC, CW), jnp.int32),
            pltpu.VMEM((GC, CW), jnp.int32),
            pltpu.VMEM((CW, DH), jnp.float32),
            pltpu.VMEM((CW, DH), jnp.float32),
            pltpu.VMEM((CW, DH), jnp.float32),
            pltpu.SemaphoreType.DMA,
            pltpu.SemaphoreType.DMA,
            pltpu.SemaphoreType.DMA,
            pltpu.SemaphoreType.DMA,
            pltpu.SemaphoreType.DMA,
            pltpu.SemaphoreType.DMA,
            pltpu.VMEM_SHARED((N_PAD, DH), jnp.float32),
        ],
    )(p_lo, p_hi, src_r, dst_r)


def _mm_body(x_ref, w_ref, h_ref):
    h_ref[...] = jnp.dot(x_ref[...], w_ref[...],
                         preferred_element_type=jnp.float32)


def _matmul0(xp, W0):
    return pl.pallas_call(
        _mm_body,
        grid=(NS,),
        in_specs=[
            pl.BlockSpec((ROWS, D), lambda i: (i, 0)),
            pl.BlockSpec((D, D), lambda i: (0, 0)),
        ],
        out_specs=pl.BlockSpec((ROWS, D), lambda i: (i, 0)),
        out_shape=jax.ShapeDtypeStruct((N_PAD, D), jnp.float32),
    )(xp, W0)


def _p0_body(h_ref, d0_ref, d1_ref, plo_ref, phi_ref, dis_ref):
    deg = d0_ref[...][:, :1] + d1_ref[...][:, :1] + 1.0
    dis = lax.rsqrt(deg)
    p = h_ref[...] * dis
    plo_ref[...] = p[:, :DH]
    phi_ref[...] = p[:, DH:]
    dis_ref[...] = jnp.broadcast_to(dis, (ROWS, 16))


def _pass1(h0, deg0, deg1):
    return pl.pallas_call(
        _p0_body,
        grid=(NS,),
        in_specs=[
            pl.BlockSpec((ROWS, D), lambda i: (i, 0)),
            pl.BlockSpec((ROWS, DH), lambda i: (i, 0)),
            pl.BlockSpec((ROWS, DH), lambda i: (i, 0)),
        ],
        out_specs=[
            pl.BlockSpec((ROWS, DH), lambda i: (i, 0)),
            pl.BlockSpec((ROWS, DH), lambda i: (i, 0)),
            pl.BlockSpec((ROWS, 16), lambda i: (i, 0)),
        ],
        out_shape=[
            jax.ShapeDtypeStruct((N_PAD, DH), jnp.float32),
            jax.ShapeDtypeStruct((N_PAD, DH), jnp.float32),
            jax.ShapeDtypeStruct((N_PAD, 16), jnp.float32),
        ],
    )(h0, deg0, deg1)


def _p1_body(slo_ref, shi_ref, dis_ref, b_ref, w_ref, plo_ref, phi_ref):
    s = jnp.concatenate([slo_ref[...], shi_ref[...]], axis=1)
    dis = dis_ref[...][:, :1]
    h = jnp.maximum(s * dis + b_ref[...], 0.0)
    p = jnp.dot(h, w_ref[...], preferred_element_type=jnp.float32) * dis
    plo_ref[...] = p[:, :DH]
    phi_ref[...] = p[:, DH:]


def _pass2(s_lo, s_hi, dis16, b0, W1):
    return pl.pallas_call(
        _p1_body,
        grid=(NS,),
        in_specs=[
            pl.BlockSpec((ROWS, DH), lambda i: (i, 0)),
            pl.BlockSpec((ROWS, DH), lambda i: (i, 0)),
            pl.BlockSpec((ROWS, 16), lambda i: (i, 0)),
            pl.BlockSpec((1, D), lambda i: (0, 0)),
            pl.BlockSpec((D, D), lambda i: (0, 0)),
        ],
        out_specs=[
            pl.BlockSpec((ROWS, DH), lambda i: (i, 0)),
            pl.BlockSpec((ROWS, DH), lambda i: (i, 0)),
        ],
        out_shape=[
            jax.ShapeDtypeStruct((N_PAD, DH), jnp.float32),
            jax.ShapeDtypeStruct((N_PAD, DH), jnp.float32),
        ],
    )(s_lo, s_hi, dis16, b0, W1)


def _head_body(slo_ref, shi_ref, dis_ref, b_ref, pw_ref, lw_ref, lb_ref,
               out_ref):
    s = jnp.concatenate([slo_ref[...], shi_ref[...]], axis=1)
    dis = dis_ref[...][:, :1]
    h = jnp.maximum(s * dis + b_ref[...], 0.0)
    row = lax.broadcasted_iota(jnp.int32, (N_PAD, 1), 0)
    valid = row < N
    h = jnp.where(valid, h, 0.0)

    pw = pw_ref[...]
    wn = jnp.sqrt(jnp.sum(pw * pw))
    raw = lax.dot_general(h, pw, (((1,), (1,)), ((), ())),
                          preferred_element_type=jnp.float32)
    score = jnp.tanh(raw / wn)
    score_m = jnp.where(valid, score, -2.0)

    bits = lax.bitcast_convert_type(score_m, jnp.uint32)
    top = jnp.uint32(0x80000000)
    ukey = jnp.where(bits >= top, ~bits, bits | top)

    def bit_step(i, t):
        cand = t | (top >> i)
        cnt = jnp.sum(jnp.where(ukey >= cand, 1, 0))
        return jnp.where(cnt >= K, cand, t)

    t = lax.fori_loop(0, 32, bit_step, jnp.uint32(0))
    cnt_gt = jnp.sum(jnp.where(ukey > t, 1, 0))
    cnt_eq = jnp.sum(jnp.where(ukey == t, 1, 0))
    frac = (K - cnt_gt).astype(jnp.float32) / jnp.maximum(cnt_eq, 1).astype(jnp.float32)
    w = jnp.where(ukey > t, score, jnp.where(ukey == t, score * frac, 0.0))

    ws = lax.dot_general(w, h, (((0,), (0,)), ((), ())),
                         preferred_element_type=jnp.float32)
    pooled = ws * (1.0 / K)
    out = jnp.dot(pooled, lw_ref[...],
                  preferred_element_type=jnp.float32) + lb_ref[...]
    m = jnp.max(out, axis=1, keepdims=True)
    z = out - m
    lse = jnp.log(jnp.sum(jnp.exp(z), axis=1, keepdims=True))
    out_ref[...] = z - lse


def _head(s_lo, s_hi, dis16, b1, pool_w, lin_w, lin_b):
    return pl.pallas_call(
        _head_body,
        grid=(1,),
        in_specs=[
            pl.BlockSpec((N_PAD, DH), lambda i: (0, 0)),
            pl.BlockSpec((N_PAD, DH), lambda i: (0, 0)),
            pl.BlockSpec((N_PAD, 16), lambda i: (0, 0)),
            pl.BlockSpec((1, D), lambda i: (0, 0)),
            pl.BlockSpec((1, D), lambda i: (0, 0)),
            pl.BlockSpec((D, D), lambda i: (0, 0)),
            pl.BlockSpec((1, D), lambda i: (0, 0)),
        ],
        out_specs=pl.BlockSpec((1, D), lambda i: (0, 0)),
        out_shape=jax.ShapeDtypeStruct((1, D), jnp.float32),
    )(s_lo, s_hi, dis16, b1, pool_w, lin_w, lin_b)


def kernel(x, edge_index, batch, W0, b0, W1, b1, pool_w, lin_w, lin_b):
    src = edge_index[0].astype(jnp.int32)
    dst = edge_index[1].astype(jnp.int32)
    src_r = src.reshape(NS * NG, GC, CW)
    dst_r = dst.reshape(NS * NG, GC, CW)
    dst_w = dst.reshape(NC * NS, NCHD, CWD)

    xp = jnp.zeros((N_PAD, D), jnp.float32).at[:N].set(x)
    ones_d = jnp.ones((CWD, DH), jnp.float32)
    zeros_d = jnp.zeros((ROWS, DH), jnp.float32)

    h0 = _matmul0(xp, W0)
    deg0, deg1 = _deg_kernel(dst_w, ones_d, zeros_d)
    p0_lo, p0_hi, dis16 = _pass1(h0, deg0, deg1)
    s0_lo, s0_hi = _agg_kernel(p0_lo, p0_hi, src_r, dst_r)
    p1_lo, p1_hi = _pass2(s0_lo, s0_hi, dis16, b0.reshape(1, D), W1)
    s1_lo, s1_hi = _agg_kernel(p1_lo, p1_hi, src_r, dst_r)
    return _head(s1_lo, s1_hi, dis16, b1.reshape(1, D),
                 pool_w.reshape(1, D), lin_w, lin_b.reshape(1, D))


# speedup vs baseline: 1.0116x; 1.0116x over previous
import functools

import jax
import jax.numpy as jnp
from jax import lax
from jax.experimental import pallas as pl
from jax.experimental.pallas import tpu as pltpu, tpu_sc as plsc

N = 10000
E = 160000
D = 256
DH = 128
NC, NS = 2, 16
N_PAD = 10240
ROWS = N_PAD // NS
EPT = E // NS
CW = 80
NCH = EPT // CW
GC = 25
NG = NCH // GC
CWD = 80
NCHD = 63
EPW = NCHD * CWD
K = (N + 1) // 2


def _sc_mesh():
    return plsc.VectorSubcoreMesh(core_axis_name="c", subcore_axis_name="s",
                                  num_cores=NC, num_subcores=NS)


def _deg_body(dst_hbm, ones_hbm, zeros_hbm, out0_hbm, out1_hbm,
              dst_ref, ones_ref, sem, acc):
    c = lax.axis_index("c")
    s = lax.axis_index("s")
    w = c * NS + s
    pltpu.sync_copy(dst_hbm.at[w], dst_ref)
    pltpu.sync_copy(ones_hbm, ones_ref)
    pltpu.sync_copy(zeros_hbm, acc.at[pl.ds(s * ROWS, ROWS)])
    plsc.subcore_barrier()

    WIN = 8

    def step(j, carry):
        pltpu.async_copy(ones_ref, acc.at[dst_ref.at[j]], sem, add=True)

        @pl.when(j >= WIN)
        def _():
            pltpu.make_async_copy(ones_ref, acc.at[dst_ref.at[j - WIN]],
                                  sem).wait()

        return carry

    lax.fori_loop(0, NCHD, step, 0)

    def drain(j, carry):
        pltpu.make_async_copy(ones_ref, acc.at[dst_ref.at[j]], sem).wait()
        return carry

    lax.fori_loop(0, WIN, drain, 0)
    plsc.subcore_barrier()

    @pl.when(c == 0)
    def _():
        pltpu.sync_copy(acc.at[pl.ds(s * ROWS, ROWS)],
                        out0_hbm.at[pl.ds(s * ROWS, ROWS)])

    @pl.when(c == 1)
    def _():
        pltpu.sync_copy(acc.at[pl.ds(s * ROWS, ROWS)],
                        out1_hbm.at[pl.ds(s * ROWS, ROWS)])


def _deg_kernel(dst_w, ones_d, zeros_d):
    return pl.kernel(
        _deg_body,
        out_type=(jax.ShapeDtypeStruct((N_PAD, DH), jnp.float32),
                  jax.ShapeDtypeStruct((N_PAD, DH), jnp.float32)),
        mesh=_sc_mesh(),
        scratch_types=[
            pltpu.VMEM((NCHD, CWD), jnp.int32),
            pltpu.VMEM((CWD, DH), jnp.float32),
            pltpu.SemaphoreType.DMA,
            pltpu.VMEM_SHARED((N_PAD, DH), jnp.float32),
        ],
    )(dst_w, ones_d, zeros_d)


def _agg_body(plo_hbm, phi_hbm, src_hbm, dst_hbm, out_lo_hbm, out_hi_hbm,
              src_ref, dst_ref, buf0, buf1, gsem0, gsem1, acc):
    c = lax.axis_index("c")
    s = lax.axis_index("s")
    bufs = (buf0, buf1)
    gsems = (gsem0, gsem1)

    def run(p_hbm, out_hbm):
        pltpu.sync_copy(p_hbm.at[pl.ds(s * ROWS, ROWS)],
                        acc.at[pl.ds(s * ROWS, ROWS)])
        plsc.subcore_barrier()

        def fire_g(l, b):
            pltpu.async_copy(p_hbm.at[src_ref.at[l]], bufs[b], gsems[b])

        def wait_g(b):
            pltpu.make_async_copy(p_hbm.at[pl.ds(0, CW)], bufs[b],
                                  gsems[b]).wait()

        def group(g, carry):
            pltpu.sync_copy(src_hbm.at[s * NG + g], src_ref)
            pltpu.sync_copy(dst_hbm.at[s * NG + g], dst_ref)
            fire_g(0, 0)

            def pair(j, c2):
                a = 2 * j
                fire_g(a + 1, 1)
                wait_g(0)
                pltpu.sync_copy(buf0, acc.at[dst_ref.at[a]], add=True)
                fire_g(a + 2, 0)
                wait_g(1)
                pltpu.sync_copy(buf1, acc.at[dst_ref.at[a + 1]], add=True)
                return c2

            lax.fori_loop(0, (GC - 1) // 2, pair, 0)
            wait_g(0)
            pltpu.sync_copy(buf0, acc.at[dst_ref.at[GC - 1]], add=True)
            return carry

        lax.fori_loop(0, NG, group, 0)
        plsc.subcore_barrier()
        pltpu.sync_copy(acc.at[pl.ds(s * ROWS, ROWS)],
                        out_hbm.at[pl.ds(s * ROWS, ROWS)])

    @pl.when(c == 0)
    def _():
        run(plo_hbm, out_lo_hbm)

    @pl.when(c == 1)
    def _():
        run(phi_hbm, out_hi_hbm)


def _agg_kernel(p_lo, p_hi, src_r, dst_r):
    return pl.kernel(
        _agg_body,
        out_type=(jax.ShapeDtypeStruct((N_PAD, DH), jnp.float32),
                  jax.ShapeDtypeStruct((N_PAD, DH), jnp.float32)),
        mesh=_sc_mesh(),
        scratch_types=[
            pltpu.VMEM((GC, CW), jnp.int32),
            pltpu.VMEM((GC, CW), jnp.int32),
            pltpu.VMEM((CW, DH), jnp.float32),
            pltpu.VMEM((CW, DH), jnp.float32),
            pltpu.SemaphoreType.DMA,
            pltpu.SemaphoreType.DMA,
            pltpu.VMEM_SHARED((N_PAD, DH), jnp.float32),
        ],
    )(p_lo, p_hi, src_r, dst_r)


def _mm_body(x_ref, w_ref, h_ref):
    h_ref[...] = jnp.dot(x_ref[...], w_ref[...],
                         preferred_element_type=jnp.float32)


def _matmul0(x, W0):
    return pl.pallas_call(
        _mm_body,
        grid=(NS,),
        in_specs=[
            pl.BlockSpec((ROWS, D), lambda i: (i, 0)),
            pl.BlockSpec((D, D), lambda i: (0, 0)),
        ],
        out_specs=pl.BlockSpec((ROWS, D), lambda i: (i, 0)),
        out_shape=jax.ShapeDtypeStruct((N_PAD, D), jnp.float32),
    )(x, W0)


def _p0_body(h_ref, d0_ref, d1_ref, plo_ref, phi_ref, dis_ref):
    deg = d0_ref[...][:, :1] + d1_ref[...][:, :1] + 1.0
    dis = lax.rsqrt(deg)
    p = h_ref[...] * dis
    plo_ref[...] = p[:, :DH]
    phi_ref[...] = p[:, DH:]
    dis_ref[...] = jnp.broadcast_to(dis, (ROWS, 16))


def _pass1(h0, deg0, deg1):
    return pl.pallas_call(
        _p0_body,
        grid=(NS,),
        in_specs=[
            pl.BlockSpec((ROWS, D), lambda i: (i, 0)),
            pl.BlockSpec((ROWS, DH), lambda i: (i, 0)),
            pl.BlockSpec((ROWS, DH), lambda i: (i, 0)),
        ],
        out_specs=[
            pl.BlockSpec((ROWS, DH), lambda i: (i, 0)),
            pl.BlockSpec((ROWS, DH), lambda i: (i, 0)),
            pl.BlockSpec((ROWS, 16), lambda i: (i, 0)),
        ],
        out_shape=[
            jax.ShapeDtypeStruct((N_PAD, DH), jnp.float32),
            jax.ShapeDtypeStruct((N_PAD, DH), jnp.float32),
            jax.ShapeDtypeStruct((N_PAD, 16), jnp.float32),
        ],
    )(h0, deg0, deg1)


def _p1_body(slo_ref, shi_ref, dis_ref, b_ref, w_ref, plo_ref, phi_ref):
    s = jnp.concatenate([slo_ref[...], shi_ref[...]], axis=1)
    dis = dis_ref[...][:, :1]
    h = jnp.maximum(s * dis + b_ref[...], 0.0)
    p = jnp.dot(h, w_ref[...], preferred_element_type=jnp.float32) * dis
    plo_ref[...] = p[:, :DH]
    phi_ref[...] = p[:, DH:]


def _pass2(s_lo, s_hi, dis16, b0, W1):
    return pl.pallas_call(
        _p1_body,
        grid=(NS,),
        in_specs=[
            pl.BlockSpec((ROWS, DH), lambda i: (i, 0)),
            pl.BlockSpec((ROWS, DH), lambda i: (i, 0)),
            pl.BlockSpec((ROWS, 16), lambda i: (i, 0)),
            pl.BlockSpec((1, D), lambda i: (0, 0)),
            pl.BlockSpec((D, D), lambda i: (0, 0)),
        ],
        out_specs=[
            pl.BlockSpec((ROWS, DH), lambda i: (i, 0)),
            pl.BlockSpec((ROWS, DH), lambda i: (i, 0)),
        ],
        out_shape=[
            jax.ShapeDtypeStruct((N_PAD, DH), jnp.float32),
            jax.ShapeDtypeStruct((N_PAD, DH), jnp.float32),
        ],
    )(s_lo, s_hi, dis16, b0, W1)


def _head_body(slo_ref, shi_ref, dis_ref, b_ref, pw_ref, lw_ref, lb_ref,
               out_ref):
    s = jnp.concatenate([slo_ref[...], shi_ref[...]], axis=1)
    dis = dis_ref[...][:, :1]
    h = jnp.maximum(s * dis + b_ref[...], 0.0)
    row = lax.broadcasted_iota(jnp.int32, (N_PAD, 1), 0)
    valid = row < N
    h = jnp.where(valid, h, 0.0)

    pw = pw_ref[...]
    wn = jnp.sqrt(jnp.sum(pw * pw))
    raw = lax.dot_general(h, pw, (((1,), (1,)), ((), ())),
                          preferred_element_type=jnp.float32)
    score = jnp.tanh(raw / wn)
    score_m = jnp.where(valid, score, -2.0)
    score_l = jnp.reshape(score_m, (N_PAD // 128, 128))

    bits = lax.bitcast_convert_type(score_l, jnp.uint32)
    top = jnp.uint32(0x80000000)
    ukey = jnp.where(bits >= top, ~bits, bits | top)

    def bit_step(i, t):
        cand = t | (top >> i)
        cnt = jnp.sum(jnp.where(ukey >= cand, 1, 0))
        return jnp.where(cnt >= K, cand, t)

    t = lax.fori_loop(0, 32, bit_step, jnp.uint32(0))
    cnt_gt = jnp.sum(jnp.where(ukey > t, 1, 0))
    cnt_eq = jnp.sum(jnp.where(ukey == t, 1, 0))
    frac = (K - cnt_gt).astype(jnp.float32) / jnp.maximum(cnt_eq, 1).astype(jnp.float32)
    w_l = jnp.where(ukey > t, score_l,
                    jnp.where(ukey == t, score_l * frac, 0.0))
    w = jnp.reshape(w_l, (N_PAD, 1))

    ws = lax.dot_general(w, h, (((0,), (0,)), ((), ())),
                         preferred_element_type=jnp.float32)
    pooled = ws * (1.0 / K)
    out = jnp.dot(pooled, lw_ref[...],
                  preferred_element_type=jnp.float32) + lb_ref[...]
    m = jnp.max(out, axis=1, keepdims=True)
    z = out - m
    lse = jnp.log(jnp.sum(jnp.exp(z), axis=1, keepdims=True))
    out_ref[...] = z - lse


def _head(s_lo, s_hi, dis16, b1, pool_w, lin_w, lin_b):
    return pl.pallas_call(
        _head_body,
        grid=(1,),
        in_specs=[
            pl.BlockSpec((N_PAD, DH), lambda i: (0, 0)),
            pl.BlockSpec((N_PAD, DH), lambda i: (0, 0)),
            pl.BlockSpec((N_PAD, 16), lambda i: (0, 0)),
            pl.BlockSpec((1, D), lambda i: (0, 0)),
            pl.BlockSpec((1, D), lambda i: (0, 0)),
            pl.BlockSpec((D, D), lambda i: (0, 0)),
            pl.BlockSpec((1, D), lambda i: (0, 0)),
        ],
        out_specs=pl.BlockSpec((1, D), lambda i: (0, 0)),
        out_shape=jax.ShapeDtypeStruct((1, D), jnp.float32),
    )(s_lo, s_hi, dis16, b1, pool_w, lin_w, lin_b)


def kernel(x, edge_index, batch, W0, b0, W1, b1, pool_w, lin_w, lin_b):
    src = edge_index[0].astype(jnp.int32)
    dst = edge_index[1].astype(jnp.int32)
    src_r = src.reshape(NS * NG, GC, CW)
    dst_r = dst.reshape(NS * NG, GC, CW)
    npad_e = NC * NS * EPW - E
    pad_dst = N + 16 + (jnp.arange(npad_e, dtype=jnp.int32) % (N_PAD - N - 16))
    dst_w = jnp.concatenate([dst, pad_dst]).reshape(NC * NS, NCHD, CWD)

    ones_d = jnp.ones((CWD, DH), jnp.float32)
    zeros_d = jnp.zeros((ROWS, DH), jnp.float32)

    h0 = _matmul0(x, W0)
    deg0, deg1 = _deg_kernel(dst_w, ones_d, zeros_d)
    p0_lo, p0_hi, dis16 = _pass1(h0, deg0, deg1)
    s0_lo, s0_hi = _agg_kernel(p0_lo, p0_hi, src_r, dst_r)
    p1_lo, p1_hi = _pass2(s0_lo, s0_hi, dis16, b0.reshape(1, D), W1)
    s1_lo, s1_hi = _agg_kernel(p1_lo, p1_hi, src_r, dst_r)
    return _head(s1_lo, s1_hi, dis16, b1.reshape(1, D),
                 pool_w.reshape(1, D), lin_w, lin_b.reshape(1, D))


# speedup vs baseline: 1.0365x; 1.0246x over previous
import functools

import jax
import jax.numpy as jnp
from jax import lax
from jax.experimental import pallas as pl
from jax.experimental.pallas import tpu as pltpu, tpu_sc as plsc

N = 10000
E = 160000
D = 256
DH = 128
NC, NS = 2, 16
N_PAD = 10240
ROWS = N_PAD // NS
EPT = E // NS
CW = 80
NCH = EPT // CW
GC = 25
NG = NCH // GC
CWD = 80
NCHD = 63
EPW = NCHD * CWD
K = (N + 1) // 2


def _sc_mesh():
    return plsc.VectorSubcoreMesh(core_axis_name="c", subcore_axis_name="s",
                                  num_cores=NC, num_subcores=NS)


def _deg_body(dst_hbm, ones_hbm, zeros_hbm, out0_hbm, out1_hbm,
              dst_ref, ones_ref, sem, acc):
    c = lax.axis_index("c")
    s = lax.axis_index("s")
    w = c * NS + s
    pltpu.sync_copy(dst_hbm.at[w], dst_ref)
    pltpu.sync_copy(ones_hbm, ones_ref)
    pltpu.sync_copy(zeros_hbm, acc.at[pl.ds(s * ROWS, ROWS)])
    plsc.subcore_barrier()

    WIN = 8

    def step(j, carry):
        pltpu.async_copy(ones_ref, acc.at[dst_ref.at[j]], sem, add=True)

        @pl.when(j >= WIN)
        def _():
            pltpu.make_async_copy(ones_ref, acc.at[dst_ref.at[j - WIN]],
                                  sem).wait()

        return carry

    lax.fori_loop(0, NCHD, step, 0)

    def drain(j, carry):
        pltpu.make_async_copy(ones_ref, acc.at[dst_ref.at[j]], sem).wait()
        return carry

    lax.fori_loop(0, WIN, drain, 0)
    plsc.subcore_barrier()

    @pl.when(c == 0)
    def _():
        pltpu.sync_copy(acc.at[pl.ds(s * ROWS, ROWS)],
                        out0_hbm.at[pl.ds(s * ROWS, ROWS)])

    @pl.when(c == 1)
    def _():
        pltpu.sync_copy(acc.at[pl.ds(s * ROWS, ROWS)],
                        out1_hbm.at[pl.ds(s * ROWS, ROWS)])


def _deg_kernel(dst_w, ones_d, zeros_d):
    return pl.kernel(
        _deg_body,
        out_type=(jax.ShapeDtypeStruct((N_PAD, DH), jnp.float32),
                  jax.ShapeDtypeStruct((N_PAD, DH), jnp.float32)),
        mesh=_sc_mesh(),
        scratch_types=[
            pltpu.VMEM((NCHD, CWD), jnp.int32),
            pltpu.VMEM((CWD, DH), jnp.float32),
            pltpu.SemaphoreType.DMA,
            pltpu.VMEM_SHARED((N_PAD, DH), jnp.float32),
        ],
    )(dst_w, ones_d, zeros_d)


def _agg_body(plo_hbm, phi_hbm, src_hbm, dst_hbm, out_lo_hbm, out_hi_hbm,
              src_ref, dst_ref, buf0, buf1, gsem0, gsem1, acc):
    c = lax.axis_index("c")
    s = lax.axis_index("s")
    bufs = (buf0, buf1)
    gsems = (gsem0, gsem1)

    def run(p_hbm, out_hbm):
        pltpu.sync_copy(p_hbm.at[pl.ds(s * ROWS, ROWS)],
                        acc.at[pl.ds(s * ROWS, ROWS)])
        plsc.subcore_barrier()

        def fire_g(l, b):
            pltpu.async_copy(p_hbm.at[src_ref.at[l]], bufs[b], gsems[b])

        def wait_g(b):
            pltpu.make_async_copy(p_hbm.at[pl.ds(0, CW)], bufs[b],
                                  gsems[b]).wait()

        def group(g, carry):
            pltpu.sync_copy(src_hbm.at[s * NG + g], src_ref)
            pltpu.sync_copy(dst_hbm.at[s * NG + g], dst_ref)
            fire_g(0, 0)

            def pair(j, c2):
                a = 2 * j
                fire_g(a + 1, 1)
                wait_g(0)
                pltpu.sync_copy(buf0, acc.at[dst_ref.at[a]], add=True)
                fire_g(a + 2, 0)
                wait_g(1)
                pltpu.sync_copy(buf1, acc.at[dst_ref.at[a + 1]], add=True)
                return c2

            lax.fori_loop(0, (GC - 1) // 2, pair, 0)
            wait_g(0)
            pltpu.sync_copy(buf0, acc.at[dst_ref.at[GC - 1]], add=True)
            return carry

        lax.fori_loop(0, NG, group, 0)
        plsc.subcore_barrier()
        pltpu.sync_copy(acc.at[pl.ds(s * ROWS, ROWS)],
                        out_hbm.at[pl.ds(s * ROWS, ROWS)])

    @pl.when(c == 0)
    def _():
        run(plo_hbm, out_lo_hbm)

    @pl.when(c == 1)
    def _():
        run(phi_hbm, out_hi_hbm)


def _agg_kernel(p_lo, p_hi, src_r, dst_r):
    return pl.kernel(
        _agg_body,
        out_type=(jax.ShapeDtypeStruct((N_PAD, DH), jnp.float32),
                  jax.ShapeDtypeStruct((N_PAD, DH), jnp.float32)),
        mesh=_sc_mesh(),
        scratch_types=[
            pltpu.VMEM((GC, CW), jnp.int32),
            pltpu.VMEM((GC, CW), jnp.int32),
            pltpu.VMEM((CW, DH), jnp.float32),
            pltpu.VMEM((CW, DH), jnp.float32),
            pltpu.SemaphoreType.DMA,
            pltpu.SemaphoreType.DMA,
            pltpu.VMEM_SHARED((N_PAD, DH), jnp.float32),
        ],
    )(p_lo, p_hi, src_r, dst_r)


def _mm_body(x_ref, w_ref, h_ref):
    h_ref[...] = jnp.dot(x_ref[...], w_ref[...],
                         preferred_element_type=jnp.float32)


def _matmul0(x, W0):
    return pl.pallas_call(
        _mm_body,
        grid=(NS,),
        in_specs=[
            pl.BlockSpec((ROWS, D), lambda i: (i, 0)),
            pl.BlockSpec((D, D), lambda i: (0, 0)),
        ],
        out_specs=pl.BlockSpec((ROWS, D), lambda i: (i, 0)),
        out_shape=jax.ShapeDtypeStruct((N_PAD, D), jnp.float32),
    )(x, W0)


def _p0_body(h_ref, d0_ref, d1_ref, plo_ref, phi_ref, dis_ref):
    deg = d0_ref[...][:, :1] + d1_ref[...][:, :1] + 1.0
    dis = lax.rsqrt(deg)
    p = h_ref[...] * dis
    plo_ref[...] = p[:, :DH]
    phi_ref[...] = p[:, DH:]
    dis_ref[...] = jnp.broadcast_to(dis, (ROWS, 16))


def _pass1(h0, deg0, deg1):
    return pl.pallas_call(
        _p0_body,
        grid=(NS,),
        in_specs=[
            pl.BlockSpec((ROWS, D), lambda i: (i, 0)),
            pl.BlockSpec((ROWS, DH), lambda i: (i, 0)),
            pl.BlockSpec((ROWS, DH), lambda i: (i, 0)),
        ],
        out_specs=[
            pl.BlockSpec((ROWS, DH), lambda i: (i, 0)),
            pl.BlockSpec((ROWS, DH), lambda i: (i, 0)),
            pl.BlockSpec((ROWS, 16), lambda i: (i, 0)),
        ],
        out_shape=[
            jax.ShapeDtypeStruct((N_PAD, DH), jnp.float32),
            jax.ShapeDtypeStruct((N_PAD, DH), jnp.float32),
            jax.ShapeDtypeStruct((N_PAD, 16), jnp.float32),
        ],
    )(h0, deg0, deg1)


def _p1_body(slo_ref, shi_ref, dis_ref, b_ref, w_ref, plo_ref, phi_ref):
    s = jnp.concatenate([slo_ref[...], shi_ref[...]], axis=1)
    dis = dis_ref[...][:, :1]
    h = jnp.maximum(s * dis + b_ref[...], 0.0)
    p = jnp.dot(h, w_ref[...], preferred_element_type=jnp.float32) * dis
    plo_ref[...] = p[:, :DH]
    phi_ref[...] = p[:, DH:]


def _pass2(s_lo, s_hi, dis16, b0, W1):
    return pl.pallas_call(
        _p1_body,
        grid=(NS,),
        in_specs=[
            pl.BlockSpec((ROWS, DH), lambda i: (i, 0)),
            pl.BlockSpec((ROWS, DH), lambda i: (i, 0)),
            pl.BlockSpec((ROWS, 16), lambda i: (i, 0)),
            pl.BlockSpec((1, D), lambda i: (0, 0)),
            pl.BlockSpec((D, D), lambda i: (0, 0)),
        ],
        out_specs=[
            pl.BlockSpec((ROWS, DH), lambda i: (i, 0)),
            pl.BlockSpec((ROWS, DH), lambda i: (i, 0)),
        ],
        out_shape=[
            jax.ShapeDtypeStruct((N_PAD, DH), jnp.float32),
            jax.ShapeDtypeStruct((N_PAD, DH), jnp.float32),
        ],
    )(s_lo, s_hi, dis16, b0, W1)


def _head_body(slo_ref, shi_ref, dis_ref, b_ref, pw_ref, lw_ref, lb_ref,
               out_ref):
    s = jnp.concatenate([slo_ref[...], shi_ref[...]], axis=1)
    dis = dis_ref[...][:, :1]
    h = jnp.maximum(s * dis + b_ref[...], 0.0)
    row = lax.broadcasted_iota(jnp.int32, (N_PAD, 1), 0)
    valid = row < N
    h = jnp.where(valid, h, 0.0)

    pw = pw_ref[...]
    wn = jnp.sqrt(jnp.sum(pw * pw))
    raw = lax.dot_general(h, pw, (((1,), (1,)), ((), ())),
                          preferred_element_type=jnp.float32)
    score = jnp.tanh(raw / wn)
    score_m = jnp.where(valid, score, -2.0)

    bits = lax.bitcast_convert_type(score_m, jnp.uint32)
    top = jnp.uint32(0x80000000)
    ukey = jnp.where(bits >= top, ~bits, bits | top)

    def bit_step(i, t):
        cand = t | (top >> i)
        cnt = jnp.sum(jnp.where(ukey >= cand, 1, 0))
        return jnp.where(cnt >= K, cand, t)

    t = lax.fori_loop(0, 32, bit_step, jnp.uint32(0))
    cnt_gt = jnp.sum(jnp.where(ukey > t, 1, 0))
    cnt_eq = jnp.sum(jnp.where(ukey == t, 1, 0))
    frac = (K - cnt_gt).astype(jnp.float32) / jnp.maximum(cnt_eq, 1).astype(jnp.float32)
    w = jnp.where(ukey > t, score, jnp.where(ukey == t, score * frac, 0.0))

    ws = lax.dot_general(w, h, (((0,), (0,)), ((), ())),
                         preferred_element_type=jnp.float32)
    pooled = ws * (1.0 / K)
    out = jnp.dot(pooled, lw_ref[...],
                  preferred_element_type=jnp.float32) + lb_ref[...]
    m = jnp.max(out, axis=1, keepdims=True)
    z = out - m
    lse = jnp.log(jnp.sum(jnp.exp(z), axis=1, keepdims=True))
    out_ref[...] = z - lse


def _head(s_lo, s_hi, dis16, b1, pool_w, lin_w, lin_b):
    return pl.pallas_call(
        _head_body,
        grid=(1,),
        in_specs=[
            pl.BlockSpec((N_PAD, DH), lambda i: (0, 0)),
            pl.BlockSpec((N_PAD, DH), lambda i: (0, 0)),
            pl.BlockSpec((N_PAD, 16), lambda i: (0, 0)),
            pl.BlockSpec((1, D), lambda i: (0, 0)),
            pl.BlockSpec((1, D), lambda i: (0, 0)),
            pl.BlockSpec((D, D), lambda i: (0, 0)),
            pl.BlockSpec((1, D), lambda i: (0, 0)),
        ],
        out_specs=pl.BlockSpec((1, D), lambda i: (0, 0)),
        out_shape=jax.ShapeDtypeStruct((1, D), jnp.float32),
    )(s_lo, s_hi, dis16, b1, pool_w, lin_w, lin_b)


def kernel(x, edge_index, batch, W0, b0, W1, b1, pool_w, lin_w, lin_b):
    src = edge_index[0].astype(jnp.int32)
    dst = edge_index[1].astype(jnp.int32)
    src_r = src.reshape(NS * NG, GC, CW)
    dst_r = dst.reshape(NS * NG, GC, CW)
    npad_e = NC * NS * EPW - E
    pad_dst = N + 16 + (jnp.arange(npad_e, dtype=jnp.int32) % (N_PAD - N - 16))
    dst_w = jnp.concatenate([dst, pad_dst]).reshape(NC * NS, NCHD, CWD)

    ones_d = jnp.ones((CWD, DH), jnp.float32)
    zeros_d = jnp.zeros((ROWS, DH), jnp.float32)

    h0 = _matmul0(x, W0)
    deg0, deg1 = _deg_kernel(dst_w, ones_d, zeros_d)
    p0_lo, p0_hi, dis16 = _pass1(h0, deg0, deg1)
    s0_lo, s0_hi = _agg_kernel(p0_lo, p0_hi, src_r, dst_r)
    p1_lo, p1_hi = _pass2(s0_lo, s0_hi, dis16, b0.reshape(1, D), W1)
    s1_lo, s1_hi = _agg_kernel(p1_lo, p1_hi, src_r, dst_r)
    return _head(s1_lo, s1_hi, dis16, b1.reshape(1, D),
                 pool_w.reshape(1, D), lin_w, lin_b.reshape(1, D))
